# baseline (speedup 1.0000x reference)
import functools

import jax
import jax.numpy as jnp
from jax import lax
from jax.experimental import pallas as pl
from jax.experimental.pallas import tpu as pltpu
from jax.experimental.pallas import tpu_sc as plsc

BATCH = 16384
NUM_FIELDS = 26
EMBED_DIM = 64
FIELD_SIZE = 100000
TOTAL = BATCH * NUM_FIELDS

_info = plsc.get_sparse_core_info()
NC = _info.num_cores
NS = _info.num_subcores
LANES = _info.num_lanes
NW = NC * NS
BROWS_PER_W = BATCH // NW

CHUNK = 256
STEPS = BROWS_PER_W // CHUNK


def _sc_gather(xt, table, f_base):
    mesh = plsc.VectorSubcoreMesh(core_axis_name="c", subcore_axis_name="s")

    @functools.partial(
        pl.kernel,
        mesh=mesh,
        compiler_params=pltpu.CompilerParams(needs_layout_passes=False),
        out_type=jax.ShapeDtypeStruct(
            (BATCH, NUM_FIELDS // 2, EMBED_DIM), jnp.float32
        ),
        scratch_types=[
            pltpu.VMEM((2, CHUNK), jnp.int32),
            pltpu.VMEM((2, CHUNK, EMBED_DIM), jnp.float32),
            pltpu.SemaphoreType.DMA,
            pltpu.SemaphoreType.DMA,
            pltpu.SemaphoreType.DMA,
            pltpu.SemaphoreType.DMA,
        ],
    )
    def k(xt_hbm, t_hbm, out_hbm, xcol, rowbuf, xsem, gsem, gsem2, osem):
        wid = lax.axis_index("s") * NC + lax.axis_index("c")
        b0 = wid * BROWS_PER_W

        def fire(f, s, buf):
            sem = gsem if buf == 0 else gsem2

            def row_body(j, carry):
                xv = xcol[buf, pl.ds(j * LANES, LANES)]
                cv = lax.max(
                    lax.min(xv, FIELD_SIZE - 1), 0
                ) + f * FIELD_SIZE
                for l in range(LANES):
                    r = jnp.squeeze(lax.slice(cv, (l,), (l + 1,)))
                    pltpu.async_copy(
                        t_hbm.at[r], rowbuf.at[buf, j * LANES + l], sem
                    )
                return carry

            lax.fori_loop(0, CHUNK // LANES, row_body, 0)

        def drain(f, s, buf):
            sem = gsem if buf == 0 else gsem2

            def wait_body(i, carry):
                pltpu.make_async_copy(
                    t_hbm.at[0], rowbuf.at[buf, 0], sem
                ).wait()
                return carry

            lax.fori_loop(0, CHUNK, wait_body, 0)
            pltpu.async_copy(
                rowbuf.at[buf],
                out_hbm.at[pl.ds(b0 + s * CHUNK, CHUNK), f],
                osem,
            ).wait()

        def field_body(f, carry):
            pltpu.async_copy(
                xt_hbm.at[f_base + f, pl.ds(b0, CHUNK)], xcol.at[0], xsem
            ).wait()
            fire(f, 0, 0)
            pltpu.async_copy(
                xt_hbm.at[f_base + f, pl.ds(b0 + CHUNK, CHUNK)], xcol.at[1], xsem
            ).wait()
            fire(f, 1, 1)
            drain(f, 0, 0)
            drain(f, 1, 1)
            return carry

        lax.fori_loop(0, NUM_FIELDS // 2, field_body, 0)

    return k(xt, table)


def kernel(x, table):
    xt = x.T
    half = NUM_FIELDS // 2 * FIELD_SIZE
    outs = [
        _sc_gather(xt, lax.slice(table, (h * half, 0), ((h + 1) * half, EMBED_DIM)), 0 if h == 0 else NUM_FIELDS // 2)
        for h in range(2)
    ]
    return jnp.concatenate(outs, axis=1)


# speedup vs baseline: 1.3233x; 1.3233x over previous
import functools

import jax
import jax.numpy as jnp
from jax import lax
from jax.experimental import pallas as pl
from jax.experimental.pallas import tpu as pltpu
from jax.experimental.pallas import tpu_sc as plsc

BATCH = 16384
NUM_FIELDS = 26
EMBED_DIM = 64
FIELD_SIZE = 100000
TOTAL = BATCH * NUM_FIELDS

_info = plsc.get_sparse_core_info()
NC = _info.num_cores
NS = _info.num_subcores
LANES = _info.num_lanes
NW = NC * NS
BROWS_PER_W = BATCH // NW

CHUNK = 256
STEPS = BROWS_PER_W // CHUNK


def _sc_gather(xt, table):
    mesh = plsc.VectorSubcoreMesh(core_axis_name="c", subcore_axis_name="s")

    @functools.partial(
        pl.kernel,
        mesh=mesh,
        compiler_params=pltpu.CompilerParams(needs_layout_passes=False),
        out_type=jax.ShapeDtypeStruct(
            (BATCH, NUM_FIELDS, EMBED_DIM), jnp.float32
        ),
        scratch_types=[
            pltpu.VMEM((2, CHUNK), jnp.int32),
            pltpu.VMEM((2, CHUNK, EMBED_DIM), jnp.float32),
            pltpu.SemaphoreType.DMA,
            pltpu.SemaphoreType.DMA,
            pltpu.SemaphoreType.DMA,
            pltpu.SemaphoreType.DMA,
        ],
    )
    def k(xt_hbm, t_hbm, out_hbm, xcol, rowbuf, xsem, gsem, gsem2, osem):
        wid = lax.axis_index("s") * NC + lax.axis_index("c")
        b0 = wid * BROWS_PER_W

        def fire(f, s, buf):
            sem = gsem if buf == 0 else gsem2

            def row_body(j, carry):
                xv = xcol[buf, pl.ds(j * LANES, LANES)]
                cv = lax.max(
                    lax.min(xv, FIELD_SIZE - 1), 0
                ) + f * FIELD_SIZE
                for l in range(LANES):
                    r = jnp.squeeze(lax.slice(cv, (l,), (l + 1,)))
                    pltpu.async_copy(
                        t_hbm.at[r], rowbuf.at[buf, j * LANES + l], sem
                    )
                return carry

            lax.fori_loop(0, CHUNK // LANES, row_body, 0)

        def drain(f, s, buf):
            sem = gsem if buf == 0 else gsem2
            pltpu.make_async_copy(
                t_hbm.at[pl.ds(0, CHUNK)], rowbuf.at[buf], sem
            ).wait()
            pltpu.async_copy(
                rowbuf.at[buf],
                out_hbm.at[pl.ds(b0 + s * CHUNK, CHUNK), f],
                osem,
            ).wait()

        def field_body(f, carry):
            pltpu.async_copy(
                xt_hbm.at[f, pl.ds(b0, CHUNK)], xcol.at[0], xsem
            ).wait()
            fire(f, 0, 0)
            pltpu.async_copy(
                xt_hbm.at[f, pl.ds(b0 + CHUNK, CHUNK)], xcol.at[1], xsem
            ).wait()
            fire(f, 1, 1)
            drain(f, 0, 0)
            drain(f, 1, 1)
            return carry

        lax.fori_loop(0, NUM_FIELDS, field_body, 0)

    return k(xt, table)


def kernel(x, table):
    out = _sc_gather(x.T, table)
    return out


# speedup vs baseline: 1.3312x; 1.0060x over previous
import functools

import jax
import jax.numpy as jnp
from jax import lax
from jax.experimental import pallas as pl
from jax.experimental.pallas import tpu as pltpu
from jax.experimental.pallas import tpu_sc as plsc

BATCH = 16384
NUM_FIELDS = 26
EMBED_DIM = 64
FIELD_SIZE = 100000
TOTAL = BATCH * NUM_FIELDS

_info = plsc.get_sparse_core_info()
NC = _info.num_cores
NS = _info.num_subcores
LANES = _info.num_lanes
NW = NC * NS
BROWS_PER_W = BATCH // NW

CHUNK = 256
STEPS = BROWS_PER_W // CHUNK


def _sc_gather(xt, table):
    mesh = plsc.VectorSubcoreMesh(core_axis_name="c", subcore_axis_name="s")

    @functools.partial(
        pl.kernel,
        mesh=mesh,
        compiler_params=pltpu.CompilerParams(needs_layout_passes=False),
        out_type=jax.ShapeDtypeStruct(
            (BATCH, NUM_FIELDS, EMBED_DIM), jnp.float32
        ),
        scratch_types=[
            pltpu.VMEM((2, CHUNK), jnp.int32),
            pltpu.VMEM((2, CHUNK, EMBED_DIM), jnp.float32),
            pltpu.SemaphoreType.DMA,
            pltpu.SemaphoreType.DMA,
            pltpu.SemaphoreType.DMA,
            pltpu.SemaphoreType.DMA,
        ],
    )
    def k(xt_hbm, t_hbm, out_hbm, xcol, rowbuf, xsem, gsem, gsem2, osem):
        wid = lax.axis_index("s") * NC + lax.axis_index("c")
        b0 = wid * BROWS_PER_W

        def fire(f, s, buf):
            sem = gsem if buf == 0 else gsem2

            def row_body(j, carry):
                xv = xcol[buf, pl.ds(j * LANES, LANES)]
                cv = lax.max(
                    lax.min(xv, FIELD_SIZE - 1), 0
                ) + f * FIELD_SIZE
                for l in range(LANES):
                    r = jnp.squeeze(lax.slice(cv, (l,), (l + 1,)))
                    pltpu.async_copy(
                        t_hbm.at[r], rowbuf.at[buf, j * LANES + l], sem
                    )
                return carry

            lax.fori_loop(0, CHUNK // LANES, row_body, 0)

        def drain(f, s, buf):
            sem = gsem if buf == 0 else gsem2
            pltpu.make_async_copy(
                t_hbm.at[pl.ds(0, CHUNK)], rowbuf.at[buf], sem
            ).wait()
            pltpu.async_copy(
                rowbuf.at[buf],
                out_hbm.at[pl.ds(b0 + s * CHUNK, CHUNK), f],
                osem,
            ).wait()

        def load_fire(f, s, buf):
            pltpu.async_copy(
                xt_hbm.at[f, pl.ds(b0 + s * CHUNK, CHUNK)], xcol.at[buf], xsem
            ).wait()
            fire(f, s, buf)

        load_fire(0, 0, 0)
        load_fire(0, 1, 1)

        def field_body(f, carry):
            drain(f, 0, 0)
            load_fire(f + 1, 0, 0)
            drain(f, 1, 1)
            load_fire(f + 1, 1, 1)
            return carry

        lax.fori_loop(0, NUM_FIELDS - 1, field_body, 0)
        drain(NUM_FIELDS - 1, 0, 0)
        drain(NUM_FIELDS - 1, 1, 1)

    return k(xt, table)


def kernel(x, table):
    out = _sc_gather(x.T, table)
    return out
